# baseline (speedup 1.0000x reference)
import functools

import jax
import jax.numpy as jnp
from jax.experimental import pallas as pl
from jax.experimental.pallas import tpu as pltpu

_L = 10


def _tc_body(spikes_ref, labels_ref, assign_ref, rates_ref, acc_ref, cnt_ref,
             *, ngrid):
    i = pl.program_id(0)

    @pl.when(i == 0)
    def _init():
        acc_ref[...] = jnp.zeros_like(acc_ref)
        cnt_ref[...] = jnp.zeros_like(cnt_ref)

    ssum = jnp.sum(spikes_ref[...], axis=1)
    lab = labels_ref[0, 0, :]
    iota_l = jax.lax.broadcasted_iota(jnp.int32, (1, _L), 1)
    oh = (lab[:, None] == iota_l).astype(jnp.float32)
    acc_ref[...] += jax.lax.dot_general(
        oh, ssum, (((0,), (0,)), ((), ())),
        precision=jax.lax.Precision.HIGHEST,
        preferred_element_type=jnp.float32)
    cnt_ref[...] += jax.lax.dot_general(
        oh, jnp.ones_like(ssum), (((0,), (0,)), ((), ())),
        precision=jax.lax.Precision.HIGHEST,
        preferred_element_type=jnp.float32)

    @pl.when(i == ngrid - 1)
    def _fini():
        acc = acc_ref[...]
        cnt = cnt_ref[...]
        mean = jnp.where(cnt > 0, acc / jnp.maximum(cnt, 1.0), 0.0)
        rates_ref[...] = mean
        best = mean[0:1, :]
        bidx = jnp.zeros_like(assign_ref)
        for l in range(1, _L):
            m = mean[l:l + 1, :]
            upd = m > best
            best = jnp.where(upd, m, best)
            bidx = jnp.where(upd, l, bidx)
        assign_ref[...] = bidx


def kernel(spikes, labels):
    B, T, N = spikes.shape
    G = 32
    Bb = B // G
    labels3 = labels.astype(jnp.int32).reshape(G, 1, Bb)
    assign2d, rates_ln = pl.pallas_call(
        functools.partial(_tc_body, ngrid=G),
        grid=(G,),
        in_specs=[
            pl.BlockSpec((Bb, T, N), lambda i: (i, 0, 0)),
            pl.BlockSpec((1, 1, Bb), lambda i: (i, 0, 0)),
        ],
        out_specs=[
            pl.BlockSpec((1, N), lambda i: (0, 0)),
            pl.BlockSpec((_L, N), lambda i: (0, 0)),
        ],
        out_shape=[
            jax.ShapeDtypeStruct((1, N), jnp.int32),
            jax.ShapeDtypeStruct((_L, N), jnp.float32),
        ],
        scratch_shapes=[
            pltpu.VMEM((_L, N), jnp.float32),
            pltpu.VMEM((_L, N), jnp.float32),
        ],
        compiler_params=pltpu.CompilerParams(
            dimension_semantics=("arbitrary",)),
    )(spikes, labels3)
    return assign2d.reshape(N), rates_ln.T


# speedup vs baseline: 1.0609x; 1.0609x over previous
import functools

import jax
import jax.numpy as jnp
from jax.experimental import pallas as pl
from jax.experimental.pallas import tpu as pltpu

_L = 10


def _tc_body(*refs, ngrid, nsplit):
    spikes_refs = refs[:nsplit]
    labels_refs = refs[nsplit:2 * nsplit]
    assign_ref, rates_ref, acc_ref, cnt_ref = refs[2 * nsplit:]
    i = pl.program_id(0)

    @pl.when(i == 0)
    def _init():
        acc_ref[...] = jnp.zeros_like(acc_ref)
        cnt_ref[...] = jnp.zeros_like(cnt_ref)

    iota_l = jax.lax.broadcasted_iota(jnp.int32, (1, _L), 1)
    for q in range(nsplit):
        ssum = jnp.sum(spikes_refs[q][...], axis=1)
        lab = labels_refs[q][0, 0, :]
        oh = (lab[:, None] == iota_l).astype(jnp.float32)
        acc_ref[...] += jax.lax.dot_general(
            oh, ssum, (((0,), (0,)), ((), ())),
            precision=jax.lax.Precision.HIGHEST,
            preferred_element_type=jnp.float32)
        cnt_ref[...] += jax.lax.dot_general(
            oh, jnp.ones_like(ssum), (((0,), (0,)), ((), ())),
            precision=jax.lax.Precision.HIGHEST,
            preferred_element_type=jnp.float32)

    @pl.when(i == ngrid - 1)
    def _fini():
        acc = acc_ref[...]
        cnt = cnt_ref[...]
        mean = jnp.where(cnt > 0, acc / jnp.maximum(cnt, 1.0), 0.0)
        rates_ref[...] = mean
        best = mean[0:1, :]
        bidx = jnp.zeros_like(assign_ref)
        for l in range(1, _L):
            m = mean[l:l + 1, :]
            upd = m > best
            best = jnp.where(upd, m, best)
            bidx = jnp.where(upd, l, bidx)
        assign_ref[...] = bidx


def kernel(spikes, labels):
    B, T, N = spikes.shape
    K = 4
    Bb = 128
    G = B // (K * Bb)
    labels3 = labels.astype(jnp.int32).reshape(B // Bb, 1, Bb)
    spike_specs = [
        pl.BlockSpec((Bb, T, N), functools.partial(
            lambda i, q: (i + q * G, 0, 0), q=q))
        for q in range(K)
    ]
    label_specs = [
        pl.BlockSpec((1, 1, Bb), functools.partial(
            lambda i, q: (i + q * G, 0, 0), q=q))
        for q in range(K)
    ]
    assign2d, rates_ln = pl.pallas_call(
        functools.partial(_tc_body, ngrid=G, nsplit=K),
        grid=(G,),
        in_specs=spike_specs + label_specs,
        out_specs=[
            pl.BlockSpec((1, N), lambda i: (0, 0)),
            pl.BlockSpec((_L, N), lambda i: (0, 0)),
        ],
        out_shape=[
            jax.ShapeDtypeStruct((1, N), jnp.int32),
            jax.ShapeDtypeStruct((_L, N), jnp.float32),
        ],
        scratch_shapes=[
            pltpu.VMEM((_L, N), jnp.float32),
            pltpu.VMEM((_L, N), jnp.float32),
        ],
        compiler_params=pltpu.CompilerParams(
            dimension_semantics=("arbitrary",)),
    )(*([spikes] * K), *([labels3] * K))
    return assign2d.reshape(N), rates_ln.T


# speedup vs baseline: 3.2071x; 3.0229x over previous
import functools

import jax
import jax.numpy as jnp
from jax.experimental import pallas as pl
from jax.experimental.pallas import tpu as pltpu

_L = 10


def _tc_body(spikes_ref, labels_ref, assign_ref, rates_ref, acc_ref, cnt_ref,
             *, ngrid):
    i = pl.program_id(0)

    @pl.when(i == 0)
    def _init():
        acc_ref[...] = jnp.zeros_like(acc_ref)
        cnt_ref[...] = jnp.zeros_like(cnt_ref)

    ssum = jnp.sum(spikes_ref[...], axis=0)
    lab = labels_ref[0, 0, :]
    iota_l = jax.lax.broadcasted_iota(jnp.int32, (1, _L), 1)
    oh = (lab[:, None] == iota_l).astype(jnp.float32)
    acc_ref[...] += jax.lax.dot_general(
        oh, ssum, (((0,), (0,)), ((), ())),
        precision=jax.lax.Precision.HIGHEST,
        preferred_element_type=jnp.float32)
    cnt_ref[...] += jax.lax.dot_general(
        oh, jnp.ones_like(ssum), (((0,), (0,)), ((), ())),
        precision=jax.lax.Precision.HIGHEST,
        preferred_element_type=jnp.float32)

    @pl.when(i == ngrid - 1)
    def _fini():
        acc = acc_ref[...]
        cnt = cnt_ref[...]
        mean = jnp.where(cnt > 0, acc / jnp.maximum(cnt, 1.0), 0.0)
        rates_ref[...] = mean
        best = mean[0:1, :]
        bidx = jnp.zeros_like(assign_ref)
        for l in range(1, _L):
            m = mean[l:l + 1, :]
            upd = m > best
            best = jnp.where(upd, m, best)
            bidx = jnp.where(upd, l, bidx)
        assign_ref[...] = bidx


def kernel(spikes, labels):
    B, T, N = spikes.shape
    st = jnp.transpose(spikes, (1, 0, 2))
    Bb = 512
    G = B // Bb
    labels3 = labels.astype(jnp.int32).reshape(G, 1, Bb)
    assign2d, rates_ln = pl.pallas_call(
        functools.partial(_tc_body, ngrid=G),
        grid=(G,),
        in_specs=[
            pl.BlockSpec((T, Bb, N), lambda i: (0, i, 0)),
            pl.BlockSpec((1, 1, Bb), lambda i: (i, 0, 0)),
        ],
        out_specs=[
            pl.BlockSpec((1, N), lambda i: (0, 0)),
            pl.BlockSpec((_L, N), lambda i: (0, 0)),
        ],
        out_shape=[
            jax.ShapeDtypeStruct((1, N), jnp.int32),
            jax.ShapeDtypeStruct((_L, N), jnp.float32),
        ],
        scratch_shapes=[
            pltpu.VMEM((_L, N), jnp.float32),
            pltpu.VMEM((_L, N), jnp.float32),
        ],
        compiler_params=pltpu.CompilerParams(
            dimension_semantics=("arbitrary",)),
    )(st, labels3)
    return assign2d.reshape(N), rates_ln.T
